# Initial kernel scaffold; baseline (speedup 1.0000x reference)
import jax
import jax.numpy as jnp
from jax.experimental import pallas as pl


def kernel(x, pe):
    raise NotImplementedError("write your pallas kernel here")


# speedup vs baseline: 2.8766x; 2.8766x over previous
import jax
import jax.numpy as jnp
from jax.experimental import pallas as pl


_LT = 512


def _qe_block(x_ref, pe_ref, o_ref):
    n = pl.program_id(1)
    row = jnp.where(n == 0, pe_ref[0, :], pe_ref[1, :])
    o_ref[...] = x_ref[...] + row[None, None, None, :]


def kernel(x, pe):
    B, N, L, K = x.shape
    grid = (B, N, L // _LT)
    return pl.pallas_call(
        _qe_block,
        grid=grid,
        in_specs=[
            pl.BlockSpec((1, 1, _LT, K), lambda b, n, l: (b, n, l, 0)),
            pl.BlockSpec((2, K), lambda b, n, l: (0, 0)),
        ],
        out_specs=pl.BlockSpec((1, 1, _LT, K), lambda b, n, l: (b, n, l, 0)),
        out_shape=jax.ShapeDtypeStruct((B, N, L, K), x.dtype),
    )(x, pe)


# speedup vs baseline: 3.1404x; 1.0917x over previous
import jax
import jax.numpy as jnp
from jax.experimental import pallas as pl


_LT = 1024


def _qe_block(x_ref, pe_ref, o_ref):
    n = pl.program_id(1)
    row = jnp.where(n == 0, pe_ref[0, :], pe_ref[1, :])
    o_ref[...] = x_ref[...] + row[None, None, None, :]


def kernel(x, pe):
    B, N, L, K = x.shape
    grid = (B, N, L // _LT)
    return pl.pallas_call(
        _qe_block,
        grid=grid,
        in_specs=[
            pl.BlockSpec((1, 1, _LT, K), lambda b, n, l: (b, n, l, 0)),
            pl.BlockSpec((2, K), lambda b, n, l: (0, 0)),
        ],
        out_specs=pl.BlockSpec((1, 1, _LT, K), lambda b, n, l: (b, n, l, 0)),
        out_shape=jax.ShapeDtypeStruct((B, N, L, K), x.dtype),
    )(x, pe)


# speedup vs baseline: 3.1807x; 1.0128x over previous
import jax
import jax.numpy as jnp
from jax.experimental import pallas as pl


_LT = 2048


def _qe_block(x_ref, pe_ref, o_ref):
    n = pl.program_id(1)
    row = jnp.where(n == 0, pe_ref[0, :], pe_ref[1, :])
    o_ref[...] = x_ref[...] + row[None, None, None, :]


def kernel(x, pe):
    B, N, L, K = x.shape
    grid = (B, N, L // _LT)
    return pl.pallas_call(
        _qe_block,
        grid=grid,
        in_specs=[
            pl.BlockSpec((1, 1, _LT, K), lambda b, n, l: (b, n, l, 0)),
            pl.BlockSpec((2, K), lambda b, n, l: (0, 0)),
        ],
        out_specs=pl.BlockSpec((1, 1, _LT, K), lambda b, n, l: (b, n, l, 0)),
        out_shape=jax.ShapeDtypeStruct((B, N, L, K), x.dtype),
    )(x, pe)
